# baseline (speedup 1.0000x reference)
import functools

import jax
import jax.numpy as jnp
from jax import lax
from jax.experimental import pallas as pl
from jax.experimental.pallas import tpu as pltpu
from jax.experimental.pallas import tpu_sc as plsc

_B = 4096
_EMB = 64
_N_GROUP = 19
_N_GRAPH = 30
_N_COLOUR = 20
_D_OUT = _EMB + _N_GROUP + _N_GRAPH + _N_COLOUR
_BN_EPS = 1e-3

_NC = 2
_NS = 16
_L = 16
_NW = _NC * _NS
_BPW = _B // _NW
_OPW = _BPW * _D_OUT
_PAD = 144

_OFF_GROUP = _EMB
_OFF_GRAPH = _EMB + _N_GROUP
_OFF_COLOUR = _EMB + _N_GROUP + _N_GRAPH


@functools.partial(
    pl.kernel,
    mesh=plsc.VectorSubcoreMesh(core_axis_name="c", subcore_axis_name="s"),
    compiler_params=pltpu.CompilerParams(
        needs_layout_passes=False, use_tc_tiling_on_sc=False),
    out_type=jax.ShapeDtypeStruct((_B * _D_OUT,), jnp.float32),
    scratch_types=[
        pltpu.VMEM((_BPW,), jnp.int32),
        pltpu.VMEM((_BPW,), jnp.int32),
        pltpu.VMEM((_BPW,), jnp.int32),
        pltpu.VMEM((_BPW,), jnp.int32),
        pltpu.VMEM((_PAD,), jnp.float32),
        pltpu.VMEM((_PAD,), jnp.float32),
        pltpu.VMEM((_BPW, _EMB), jnp.float32),
        pltpu.VMEM((_OPW,), jnp.float32),
        pltpu.SemaphoreType.DMA,
    ],
)
def _article_sc(aid_hbm, grp_hbm, gph_hbm, col_hbm, table_hbm, scale_hbm,
                shift_hbm, out_hbm, aid_v, grp_v, gph_v, col_v, scale_v,
                shift_v, rows_v, out_v, sem):
    wid = lax.axis_index("s") * _NC + lax.axis_index("c")
    base = wid * _BPW

    pltpu.sync_copy(aid_hbm.at[pl.ds(base, _BPW)], aid_v)
    gather = pltpu.async_copy(table_hbm.at[aid_v], rows_v, sem)

    pltpu.sync_copy(grp_hbm.at[pl.ds(base, _BPW)], grp_v)
    pltpu.sync_copy(gph_hbm.at[pl.ds(base, _BPW)], gph_v)
    pltpu.sync_copy(col_hbm.at[pl.ds(base, _BPW)], col_v)
    pltpu.sync_copy(scale_hbm, scale_v)
    pltpu.sync_copy(shift_hbm, shift_v)

    sh_a = shift_v[pl.ds(_OFF_GROUP, _L)]
    sh_b = shift_v[pl.ds(80, _L)]
    sh_c = shift_v[pl.ds(96, _L)]
    sh_d = shift_v[pl.ds(112, _L)]
    sh_e = shift_v[pl.ds(_D_OUT - _L, _L)]

    def init_row(r, carry):
        o = r * _D_OUT
        out_v[pl.ds(o + 64, _L)] = sh_a
        out_v[pl.ds(o + 80, _L)] = sh_b
        out_v[pl.ds(o + 96, _L)] = sh_c
        out_v[pl.ds(o + 112, _L)] = sh_d
        out_v[pl.ds(o + (_D_OUT - _L), _L)] = sh_e
        return carry

    lax.fori_loop(0, _BPW, init_row, 0)

    lane = lax.iota(jnp.int32, _L)
    for blk in range(_BPW // _L):
        rowbase = (lane + blk * _L) * _D_OUT
        for idx_ref, off in ((grp_v, _OFF_GROUP), (gph_v, _OFF_GRAPH),
                             (col_v, _OFF_COLOUR)):
            ids = idx_ref[pl.ds(blk * _L, _L)] + off
            vals = plsc.load_gather(scale_v, [ids])
            plsc.addupdate_scatter(out_v, [rowbase + ids], vals)

    gather.wait()

    emb_sc = [scale_v[pl.ds(c * _L, _L)] for c in range(_EMB // _L)]
    emb_sh = [shift_v[pl.ds(c * _L, _L)] for c in range(_EMB // _L)]

    def emb_row(r, carry):
        o = r * _D_OUT
        for c in range(_EMB // _L):
            out_v[pl.ds(o + c * _L, _L)] = (
                rows_v[r, pl.ds(c * _L, _L)] * emb_sc[c] + emb_sh[c])
        return carry

    lax.fori_loop(0, _BPW, emb_row, 0)

    pltpu.sync_copy(out_v, out_hbm.at[pl.ds(base * _D_OUT, _OPW)])


def kernel(article_id, product_group_name, graphical_appearance_name,
           perceived_colour_master_name, emb_table, gamma, beta,
           moving_mean, moving_var):
    scale = gamma * lax.rsqrt(moving_var + _BN_EPS)
    shift = beta - moving_mean * scale
    scale_p = jnp.pad(scale, (0, _PAD - _D_OUT))
    shift_p = jnp.pad(shift, (0, _PAD - _D_OUT))
    out = _article_sc(
        article_id.astype(jnp.int32),
        product_group_name.astype(jnp.int32),
        graphical_appearance_name.astype(jnp.int32),
        perceived_colour_master_name.astype(jnp.int32),
        emb_table,
        scale_p,
        shift_p,
    )
    return out.reshape(_B, _D_OUT)


# speedup vs baseline: 1.3483x; 1.3483x over previous
import functools

import jax
import jax.numpy as jnp
from jax import lax
from jax.experimental import pallas as pl
from jax.experimental.pallas import tpu as pltpu
from jax.experimental.pallas import tpu_sc as plsc

_B = 4096
_EMB = 64
_N_GROUP = 19
_N_GRAPH = 30
_N_COLOUR = 20
_D_OUT = _EMB + _N_GROUP + _N_GRAPH + _N_COLOUR
_BN_EPS = 1e-3

_NC = 2
_NS = 16
_L = 16
_NW = _NC * _NS
_BPW = _B // _NW
_OPW = _BPW * _D_OUT
_PAD = 144

_OFF_GROUP = _EMB
_OFF_GRAPH = _EMB + _N_GROUP
_OFF_COLOUR = _EMB + _N_GROUP + _N_GRAPH


@functools.partial(
    pl.kernel,
    mesh=plsc.VectorSubcoreMesh(core_axis_name="c", subcore_axis_name="s"),
    compiler_params=pltpu.CompilerParams(
        needs_layout_passes=False, use_tc_tiling_on_sc=True),
    out_type=jax.ShapeDtypeStruct((_B * _D_OUT,), jnp.float32),
    scratch_types=[
        pltpu.VMEM((_BPW,), jnp.int32),
        pltpu.VMEM((_BPW,), jnp.int32),
        pltpu.VMEM((_BPW,), jnp.int32),
        pltpu.VMEM((_BPW,), jnp.int32),
        pltpu.VMEM((_PAD,), jnp.float32),
        pltpu.VMEM((_PAD,), jnp.float32),
        pltpu.VMEM((_BPW, _EMB), jnp.float32),
        pltpu.VMEM((_OPW,), jnp.float32),
        pltpu.SemaphoreType.DMA,
    ],
)
def _article_sc(aid_hbm, grp_hbm, gph_hbm, col_hbm, table_hbm, scale_hbm,
                shift_hbm, out_hbm, aid_v, grp_v, gph_v, col_v,
                scale_v, shift_v, rows_v, out_v, sem):
    wid = lax.axis_index("s") * _NC + lax.axis_index("c")
    base = wid * _BPW

    pltpu.sync_copy(aid_hbm.at[pl.ds(base, _BPW)], aid_v)

    for blk in range(_BPW // _L):
        ids = aid_v[pl.ds(blk * _L, _L)]
        for lane in range(_L):
            r = blk * _L + lane
            pltpu.async_copy(table_hbm.at[ids[lane]], rows_v.at[r], sem)

    pltpu.sync_copy(grp_hbm.at[pl.ds(base, _BPW)], grp_v)
    pltpu.sync_copy(gph_hbm.at[pl.ds(base, _BPW)], gph_v)
    pltpu.sync_copy(col_hbm.at[pl.ds(base, _BPW)], col_v)
    pltpu.sync_copy(scale_hbm, scale_v)
    pltpu.sync_copy(shift_hbm, shift_v)

    sh_a = shift_v[pl.ds(_OFF_GROUP, _L)]
    sh_b = shift_v[pl.ds(80, _L)]
    sh_c = shift_v[pl.ds(96, _L)]
    sh_d = shift_v[pl.ds(112, _L)]
    sh_e = shift_v[pl.ds(_D_OUT - _L, _L)]

    def init_row(r, carry):
        o = r * _D_OUT
        out_v[pl.ds(o + 64, _L)] = sh_a
        out_v[pl.ds(o + 80, _L)] = sh_b
        out_v[pl.ds(o + 96, _L)] = sh_c
        out_v[pl.ds(o + 112, _L)] = sh_d
        out_v[pl.ds(o + (_D_OUT - _L), _L)] = sh_e
        return carry

    lax.fori_loop(0, _BPW, init_row, 0)

    lane = lax.iota(jnp.int32, _L)
    for blk in range(_BPW // _L):
        rowbase = (lane + blk * _L) * _D_OUT
        for idx_ref, off in ((grp_v, _OFF_GROUP), (gph_v, _OFF_GRAPH),
                             (col_v, _OFF_COLOUR)):
            ids = idx_ref[pl.ds(blk * _L, _L)] + off
            vals = plsc.load_gather(scale_v, [ids])
            plsc.addupdate_scatter(out_v, [rowbase + ids], vals)

    pltpu.make_async_copy(table_hbm.at[pl.ds(0, _BPW)], rows_v, sem).wait()

    emb_sc = [scale_v[pl.ds(c * _L, _L)] for c in range(_EMB // _L)]
    emb_sh = [shift_v[pl.ds(c * _L, _L)] for c in range(_EMB // _L)]

    def emb_row(r, carry):
        o = r * _D_OUT
        for c in range(_EMB // _L):
            out_v[pl.ds(o + c * _L, _L)] = (
                rows_v[r, pl.ds(c * _L, _L)] * emb_sc[c] + emb_sh[c])
        return carry

    lax.fori_loop(0, _BPW, emb_row, 0)

    pltpu.sync_copy(out_v, out_hbm.at[pl.ds(base * _D_OUT, _OPW)])


def kernel(article_id, product_group_name, graphical_appearance_name,
           perceived_colour_master_name, emb_table, gamma, beta,
           moving_mean, moving_var):
    scale = gamma * lax.rsqrt(moving_var + _BN_EPS)
    shift = beta - moving_mean * scale
    scale_p = jnp.pad(scale, (0, _PAD - _D_OUT))
    shift_p = jnp.pad(shift, (0, _PAD - _D_OUT))
    out = _article_sc(
        article_id.astype(jnp.int32),
        product_group_name.astype(jnp.int32),
        graphical_appearance_name.astype(jnp.int32),
        perceived_colour_master_name.astype(jnp.int32),
        emb_table,
        scale_p,
        shift_p,
    )
    return out.reshape(_B, _D_OUT)


# speedup vs baseline: 2.1408x; 1.5878x over previous
import functools

import jax
import jax.numpy as jnp
from jax import lax
from jax.experimental import pallas as pl
from jax.experimental.pallas import tpu as pltpu
from jax.experimental.pallas import tpu_sc as plsc

_B = 4096
_V = 100001
_EMB = 64
_N_GROUP = 19
_N_GRAPH = 30
_N_COLOUR = 20
_D_OUT = _EMB + _N_GROUP + _N_GRAPH + _N_COLOUR
_BN_EPS = 1e-3

_NC = 2
_NS = 16
_L = 16
_NW = _NC * _NS
_DPW = _EMB // _NW

_BLK = 512
_GRID = _B // _BLK


@functools.partial(
    pl.kernel,
    mesh=plsc.VectorSubcoreMesh(core_axis_name="c", subcore_axis_name="s"),
    compiler_params=pltpu.CompilerParams(
        needs_layout_passes=False, use_tc_tiling_on_sc=True),
    out_type=jax.ShapeDtypeStruct((_EMB, _B), jnp.float32),
    scratch_types=[
        pltpu.VMEM((_B,), jnp.int32),
        pltpu.VMEM((_V,), jnp.float32),
        pltpu.VMEM((_B,), jnp.float32),
        pltpu.SemaphoreType.DMA,
    ],
)
def _gather_sc(table_hbm, aid_hbm, out_hbm, aid_v, row_v, col_v, sem):
    wid = lax.axis_index("s") * _NC + lax.axis_index("c")

    pltpu.sync_copy(aid_hbm, aid_v)

    for k in range(_DPW):
        d = wid * _DPW + k
        pltpu.sync_copy(table_hbm.at[d], row_v)

        def gath(j, carry):
            ids = aid_v[pl.ds(j * _L, _L)]
            col_v[pl.ds(j * _L, _L)] = plsc.load_gather(row_v, [ids])
            return carry

        lax.fori_loop(0, _B // _L, gath, 0)
        pltpu.sync_copy(col_v, out_hbm.at[d])


def _assemble_tc(emb_ref, grp_ref, gph_ref, col_ref, scale_ref, shift_ref,
                 out_ref):
    e = emb_ref[...]
    grp = grp_ref[0]
    gph = gph_ref[0]
    col = col_ref[0]
    ohg = jnp.where(
        lax.broadcasted_iota(jnp.int32, (_N_GROUP, _BLK), 0) == grp,
        1.0, 0.0).astype(jnp.float32)
    ohh = jnp.where(
        lax.broadcasted_iota(jnp.int32, (_N_GRAPH, _BLK), 0) == gph,
        1.0, 0.0).astype(jnp.float32)
    ohc = jnp.where(
        lax.broadcasted_iota(jnp.int32, (_N_COLOUR, _BLK), 0) == col,
        1.0, 0.0).astype(jnp.float32)
    full_t = jnp.concatenate([e, ohg, ohh, ohc], axis=0)
    x = jnp.transpose(full_t)
    out_ref[...] = x * scale_ref[...] + shift_ref[...]


_assemble = pl.pallas_call(
    _assemble_tc,
    grid=(_GRID,),
    in_specs=[
        pl.BlockSpec((_EMB, _BLK), lambda i: (0, i)),
        pl.BlockSpec((1, 1, _BLK), lambda i: (i, 0, 0)),
        pl.BlockSpec((1, 1, _BLK), lambda i: (i, 0, 0)),
        pl.BlockSpec((1, 1, _BLK), lambda i: (i, 0, 0)),
        pl.BlockSpec((1, _D_OUT), lambda i: (0, 0)),
        pl.BlockSpec((1, _D_OUT), lambda i: (0, 0)),
    ],
    out_specs=pl.BlockSpec((_BLK, _D_OUT), lambda i: (i, 0)),
    out_shape=jax.ShapeDtypeStruct((_B, _D_OUT), jnp.float32),
)


def kernel(article_id, product_group_name, graphical_appearance_name,
           perceived_colour_master_name, emb_table, gamma, beta,
           moving_mean, moving_var):
    scale = gamma * lax.rsqrt(moving_var + _BN_EPS)
    shift = beta - moving_mean * scale
    table_t = emb_table.T
    emb_t = _gather_sc(table_t, article_id.astype(jnp.int32))
    return _assemble(
        emb_t,
        product_group_name.astype(jnp.int32).reshape(_GRID, 1, _BLK),
        graphical_appearance_name.astype(jnp.int32).reshape(_GRID, 1, _BLK),
        perceived_colour_master_name.astype(jnp.int32).reshape(_GRID, 1, _BLK),
        scale.reshape(1, _D_OUT),
        shift.reshape(1, _D_OUT),
    )


# speedup vs baseline: 2.4857x; 1.1611x over previous
import functools

import jax
import jax.numpy as jnp
from jax import lax
from jax.experimental import pallas as pl
from jax.experimental.pallas import tpu as pltpu
from jax.experimental.pallas import tpu_sc as plsc

_B = 4096
_V = 100001
_EMB = 64
_N_GROUP = 19
_N_GRAPH = 30
_N_COLOUR = 20
_D_OUT = _EMB + _N_GROUP + _N_GRAPH + _N_COLOUR
_BN_EPS = 1e-3

_NC = 2
_NS = 16
_L = 16
_NW = _NC * _NS
_DPW = _EMB // _NW

_BLK = 512
_GRID = _B // _BLK
_UNROLL = 8


@functools.partial(
    pl.kernel,
    mesh=plsc.VectorSubcoreMesh(core_axis_name="c", subcore_axis_name="s"),
    compiler_params=pltpu.CompilerParams(
        needs_layout_passes=False, use_tc_tiling_on_sc=True),
    out_type=jax.ShapeDtypeStruct((_EMB, _B), jnp.float32),
    scratch_types=[
        pltpu.VMEM((_B,), jnp.int32),
        pltpu.VMEM((_V,), jnp.float32),
        pltpu.VMEM((_B,), jnp.float32),
        pltpu.SemaphoreType.DMA,
    ],
)
def _gather_sc(table_hbm, aid_hbm, out_hbm, aid_v, row_v, col_v, sem):
    wid = lax.axis_index("s") * _NC + lax.axis_index("c")

    pltpu.sync_copy(aid_hbm, aid_v)

    for k in range(_DPW):
        d = wid * _DPW + k
        pltpu.sync_copy(table_hbm.at[d], row_v)

        def gath(j, carry):
            for u in range(_UNROLL):
                o = (j * _UNROLL + u) * _L
                ids = aid_v[pl.ds(o, _L)]
                col_v[pl.ds(o, _L)] = plsc.load_gather(row_v, [ids])
            return carry

        lax.fori_loop(0, _B // _L // _UNROLL, gath, 0)
        pltpu.sync_copy(col_v, out_hbm.at[d])


def _assemble_tc(emb_ref, grp_ref, gph_ref, col_ref, scale_ref, shift_ref,
                 out_ref):
    e = emb_ref[...]
    grp = grp_ref[0]
    gph = gph_ref[0]
    col = col_ref[0]
    ohg = jnp.where(
        lax.broadcasted_iota(jnp.int32, (_N_GROUP, _BLK), 0) == grp,
        1.0, 0.0).astype(jnp.float32)
    ohh = jnp.where(
        lax.broadcasted_iota(jnp.int32, (_N_GRAPH, _BLK), 0) == gph,
        1.0, 0.0).astype(jnp.float32)
    ohc = jnp.where(
        lax.broadcasted_iota(jnp.int32, (_N_COLOUR, _BLK), 0) == col,
        1.0, 0.0).astype(jnp.float32)
    full_t = jnp.concatenate([e, ohg, ohh, ohc], axis=0)
    out_ref[...] = full_t * scale_ref[...] + shift_ref[...]


_assemble = pl.pallas_call(
    _assemble_tc,
    grid=(_GRID,),
    in_specs=[
        pl.BlockSpec((_EMB, _BLK), lambda i: (0, i)),
        pl.BlockSpec((1, 1, _BLK), lambda i: (i, 0, 0)),
        pl.BlockSpec((1, 1, _BLK), lambda i: (i, 0, 0)),
        pl.BlockSpec((1, 1, _BLK), lambda i: (i, 0, 0)),
        pl.BlockSpec((_D_OUT, 1), lambda i: (0, 0)),
        pl.BlockSpec((_D_OUT, 1), lambda i: (0, 0)),
    ],
    out_specs=pl.BlockSpec((_D_OUT, _BLK), lambda i: (0, i)),
    out_shape=jax.ShapeDtypeStruct((_D_OUT, _B), jnp.float32),
)


def kernel(article_id, product_group_name, graphical_appearance_name,
           perceived_colour_master_name, emb_table, gamma, beta,
           moving_mean, moving_var):
    scale = gamma * lax.rsqrt(moving_var + _BN_EPS)
    shift = beta - moving_mean * scale
    table_t = emb_table.T
    emb_t = _gather_sc(table_t, article_id.astype(jnp.int32))
    out_t = _assemble(
        emb_t,
        product_group_name.astype(jnp.int32).reshape(_GRID, 1, _BLK),
        graphical_appearance_name.astype(jnp.int32).reshape(_GRID, 1, _BLK),
        perceived_colour_master_name.astype(jnp.int32).reshape(_GRID, 1, _BLK),
        scale.reshape(_D_OUT, 1),
        shift.reshape(_D_OUT, 1),
    )
    return out_t.T


# speedup vs baseline: 2.5935x; 1.0434x over previous
import functools

import jax
import jax.numpy as jnp
from jax import lax
from jax.experimental import pallas as pl
from jax.experimental.pallas import tpu as pltpu
from jax.experimental.pallas import tpu_sc as plsc

_B = 4096
_V = 100001
_EMB = 64
_N_GROUP = 19
_N_GRAPH = 30
_N_COLOUR = 20
_D_OUT = _EMB + _N_GROUP + _N_GRAPH + _N_COLOUR
_D_OH = _D_OUT - _EMB
_BN_EPS = 1e-3

_NC = 2
_NS = 16
_L = 16
_NW = _NC * _NS
_DPW = _EMB // _NW
_APW = _B // _NW
_UNROLL = 8

_OFF_GROUP = _EMB
_OFF_GRAPH = _EMB + _N_GROUP
_OFF_COLOUR = _OFF_GRAPH + _N_GRAPH
_PAD = 144


@functools.partial(
    pl.kernel,
    mesh=plsc.VectorSubcoreMesh(core_axis_name="c", subcore_axis_name="s"),
    compiler_params=pltpu.CompilerParams(
        needs_layout_passes=False, use_tc_tiling_on_sc=True),
    out_type=jax.ShapeDtypeStruct((_D_OUT, _B), jnp.float32),
    scratch_types=[
        pltpu.VMEM((_B,), jnp.int32),
        pltpu.VMEM((_APW,), jnp.int32),
        pltpu.VMEM((_APW,), jnp.int32),
        pltpu.VMEM((_APW,), jnp.int32),
        pltpu.VMEM((_PAD,), jnp.float32),
        pltpu.VMEM((_PAD,), jnp.float32),
        pltpu.VMEM((_V,), jnp.float32),
        pltpu.VMEM((_B,), jnp.float32),
        pltpu.VMEM((_D_OH, _APW), jnp.float32),
        pltpu.SemaphoreType.DMA,
        pltpu.SemaphoreType.DMA,
    ],
)
def _article_sc(table_hbm, aid_hbm, grp_hbm, gph_hbm, col_hbm, scale_hbm,
                shift_hbm, ohinit_hbm, out_hbm, aid_v, grp_v, gph_v, colr_v,
                scale_v, shift_v, row_v, col_v, oh_v, sem, osem):
    wid = lax.axis_index("s") * _NC + lax.axis_index("c")
    abase = wid * _APW

    d0 = wid * _DPW
    rowcp = pltpu.async_copy(table_hbm.at[d0], row_v, sem)

    pltpu.sync_copy(aid_hbm, aid_v)
    pltpu.sync_copy(grp_hbm.at[pl.ds(abase, _APW)], grp_v)
    pltpu.sync_copy(gph_hbm.at[pl.ds(abase, _APW)], gph_v)
    pltpu.sync_copy(col_hbm.at[pl.ds(abase, _APW)], colr_v)
    pltpu.sync_copy(scale_hbm, scale_v)
    pltpu.sync_copy(shift_hbm, shift_v)
    pltpu.sync_copy(ohinit_hbm, oh_v)

    lane = lax.iota(jnp.int32, _L)
    for blk in range(_APW // _L):
        cols = lane + blk * _L
        for idx_ref, off in ((grp_v, _OFF_GROUP), (gph_v, _OFF_GRAPH),
                             (colr_v, _OFF_COLOUR)):
            ids = idx_ref[pl.ds(blk * _L, _L)] + (off - _EMB)
            vals = plsc.load_gather(scale_v, [ids + _EMB])
            plsc.addupdate_scatter(oh_v, [ids, cols], vals)
    ohcp = pltpu.async_copy(
        oh_v, out_hbm.at[pl.ds(_EMB, _D_OH), pl.ds(abase, _APW)], osem)

    for k in range(_DPW):
        d = d0 + k
        rowcp.wait()
        dsplat = jnp.full((_L,), d, jnp.int32)
        sc_d = plsc.load_gather(scale_v, [dsplat])
        sh_d = plsc.load_gather(shift_v, [dsplat])

        def gath(j, carry):
            for u in range(_UNROLL):
                o = (j * _UNROLL + u) * _L
                ids = aid_v[pl.ds(o, _L)]
                col_v[pl.ds(o, _L)] = (
                    plsc.load_gather(row_v, [ids]) * sc_d + sh_d)
            return carry

        lax.fori_loop(0, _B // _L // _UNROLL, gath, 0)
        if k + 1 < _DPW:
            rowcp = pltpu.async_copy(table_hbm.at[d0 + k + 1], row_v, sem)
        pltpu.sync_copy(col_v, out_hbm.at[d])

    ohcp.wait()


def kernel(article_id, product_group_name, graphical_appearance_name,
           perceived_colour_master_name, emb_table, gamma, beta,
           moving_mean, moving_var):
    scale = gamma * lax.rsqrt(moving_var + _BN_EPS)
    shift = beta - moving_mean * scale
    scale_p = jnp.pad(scale, (0, _PAD - _D_OUT))
    shift_p = jnp.pad(shift, (0, _PAD - _D_OUT))
    ohinit = jnp.broadcast_to(shift[_EMB:, None], (_D_OH, _APW))
    table_t = emb_table.T
    out_t = _article_sc(
        table_t,
        article_id.astype(jnp.int32),
        product_group_name.astype(jnp.int32),
        graphical_appearance_name.astype(jnp.int32),
        perceived_colour_master_name.astype(jnp.int32),
        scale_p,
        shift_p,
        ohinit,
    )
    return out_t.T
